# Initial kernel scaffold; baseline (speedup 1.0000x reference)
import functools

import jax
import jax.numpy as jnp
from jax import lax
from jax.experimental import pallas as pl
from jax.experimental.pallas import tpu as pltpu
from jax.experimental.pallas import tpu_sc as plsc

N_NODES = 10000
N_EDGES = 160000
D_IN = 256
D_OUT = 256
HALF = 128

NC = 2
NS = 16
LANES = 16

CHUNK = 128
N_CHUNKS = N_EDGES // CHUNK
CHUNKS_PER_TILE = -(-N_CHUNKS // NS)
ROWS_PER_TILE = N_NODES // NS


def _sc_agg_body(x2_hbm, ei_hbm, ev_hbm, out_hbm,
                 src_v, dst_v, ev_v, rows_v, acc_sh, sem):
    c = lax.axis_index("c")
    s = lax.axis_index("s")

    zero16 = jnp.zeros((LANES,), jnp.float32)

    def zrow(r, carry):
        for p in range(HALF // LANES):
            rows_v[r, pl.ds(p * LANES, LANES)] = zero16
        return carry

    lax.fori_loop(0, CHUNK, zrow, 0)
    for k in range(5):
        pltpu.sync_copy(rows_v.at[pl.ds(0, 125)],
                        acc_sh.at[pl.ds(s * ROWS_PER_TILE + k * 125, 125)])
    plsc.subcore_barrier()

    def chunk_body(i, carry):
        cid = i * NS + s

        @pl.when(cid < N_CHUNKS)
        def _():
            base = cid * CHUNK
            pltpu.sync_copy(ei_hbm.at[1, pl.ds(base, CHUNK)], src_v)
            pltpu.sync_copy(ei_hbm.at[0, pl.ds(base, CHUNK)], dst_v)
            pltpu.sync_copy(ev_hbm.at[pl.ds(base, CHUNK)], ev_v)
            for p in range(CHUNK // LANES):
                sl = pl.ds(p * LANES, LANES)
                src_v[sl] = src_v[sl] * 2 + c
            pltpu.async_copy(x2_hbm.at[src_v], rows_v, sem).wait()

            def srow(g, inner):
                e = ev_v[g]
                for p in range(HALF // LANES):
                    sl = pl.ds(p * LANES, LANES)
                    rows_v[g, sl] = rows_v[g, sl] * e
                return inner

            lax.fori_loop(0, CHUNK, srow, 0)
            pltpu.sync_copy(rows_v, acc_sh.at[dst_v], add=True)

        return carry

    lax.fori_loop(0, CHUNKS_PER_TILE, chunk_body, 0)
    plsc.subcore_barrier()
    pltpu.sync_copy(acc_sh.at[pl.ds(s * ROWS_PER_TILE, ROWS_PER_TILE)],
                    out_hbm.at[c, pl.ds(s * ROWS_PER_TILE, ROWS_PER_TILE)])


_sc_agg = functools.partial(
    pl.kernel,
    out_type=jax.ShapeDtypeStruct((NC, N_NODES, HALF), jnp.float32),
    mesh=plsc.VectorSubcoreMesh(core_axis_name="c", subcore_axis_name="s"),
    scratch_types=[
        pltpu.VMEM((CHUNK,), jnp.int32),
        pltpu.VMEM((CHUNK,), jnp.int32),
        pltpu.VMEM((CHUNK,), jnp.float32),
        pltpu.VMEM((CHUNK, HALF), jnp.float32),
        pltpu.VMEM_SHARED((N_NODES, HALF), jnp.float32),
        pltpu.SemaphoreType.DMA,
    ],
)(_sc_agg_body)


def _mm_body(a0_ref, a1_ref, w0_ref, w1_ref, o_ref):
    acc = jnp.dot(a0_ref[...], w0_ref[...],
                  preferred_element_type=jnp.float32,
                  precision=lax.Precision.HIGHEST)
    acc = acc + jnp.dot(a1_ref[...], w1_ref[...],
                        preferred_element_type=jnp.float32,
                        precision=lax.Precision.HIGHEST)
    o_ref[...] = jnp.maximum(acc, 0.0)


M_BLK = 1000


def _mm_relu(agg2, w):
    return pl.pallas_call(
        _mm_body,
        grid=(N_NODES // M_BLK,),
        in_specs=[
            pl.BlockSpec((M_BLK, HALF), lambda i: (i, 0)),
            pl.BlockSpec((M_BLK, HALF), lambda i: (i, 0)),
            pl.BlockSpec((HALF, D_OUT), lambda i: (0, 0)),
            pl.BlockSpec((HALF, D_OUT), lambda i: (0, 0)),
        ],
        out_specs=pl.BlockSpec((M_BLK, D_OUT), lambda i: (i, 0)),
        out_shape=jax.ShapeDtypeStruct((N_NODES, D_OUT), jnp.float32),
    )(agg2[0], agg2[1], w[:HALF], w[HALF:])


def kernel(x, edge_index, edge_values, W):
    x2 = x.reshape(2 * N_NODES, HALF)
    agg2 = _sc_agg(x2, edge_index, edge_values)
    return _mm_relu(agg2, W)


# speedup vs baseline: 3.4715x; 3.4715x over previous
import functools

import jax
import jax.numpy as jnp
from jax import lax
from jax.experimental import pallas as pl
from jax.experimental.pallas import tpu as pltpu
from jax.experimental.pallas import tpu_sc as plsc

N_NODES = 10000
N_EDGES = 160000
D_IN = 256
D_OUT = 256
HALF = 128

NC = 2
NS = 16
LANES = 16

CHUNK = 128
N_CHUNKS = N_EDGES // CHUNK
CHUNKS_PER_TILE = -(-N_CHUNKS // NS)
ROWS_MAIN = 624


def _sc_agg_body(x2_hbm, ei_hbm, ev_hbm, out_hbm,
                 src_v, dst_v, ev_v, rows_v, acc_sh, sem):
    c = lax.axis_index("c")
    s = lax.axis_index("s")

    zero16 = jnp.zeros((LANES,), jnp.float32)

    def zrow(r, carry):
        for p in range(HALF // LANES):
            rows_v[r, pl.ds(p * LANES, LANES)] = zero16
        return carry

    lax.fori_loop(0, CHUNK, zrow, 0)
    start = pl.multiple_of(s * ROWS_MAIN, 8)
    for k in range(4):
        pltpu.sync_copy(rows_v,
                        acc_sh.at[pl.ds(start + k * CHUNK, CHUNK)])
    pltpu.sync_copy(rows_v.at[pl.ds(0, ROWS_MAIN - 4 * CHUNK)],
                    acc_sh.at[pl.ds(start + 4 * CHUNK, ROWS_MAIN - 4 * CHUNK)])

    @pl.when(s == NS - 1)
    def _():
        pltpu.sync_copy(rows_v.at[pl.ds(0, N_NODES - NS * ROWS_MAIN)],
                        acc_sh.at[pl.ds(NS * ROWS_MAIN, N_NODES - NS * ROWS_MAIN)])

    plsc.subcore_barrier()

    def chunk_body(i, carry):
        cid = i * NS + s

        @pl.when(cid < N_CHUNKS)
        def _():
            base = cid * CHUNK
            pltpu.sync_copy(ei_hbm.at[1, pl.ds(base, CHUNK)], src_v)
            pltpu.sync_copy(ei_hbm.at[0, pl.ds(base, CHUNK)], dst_v)
            pltpu.sync_copy(ev_hbm.at[pl.ds(base, CHUNK)], ev_v)
            for p in range(CHUNK // LANES):
                sl = pl.ds(p * LANES, LANES)
                src_v[sl] = src_v[sl] * 2 + c
            pltpu.async_copy(x2_hbm.at[src_v], rows_v, sem).wait()

            def srow(gg, inner):
                evvec = ev_v[pl.ds(gg * LANES, LANES)]
                for l in range(LANES):
                    e = evvec[l]
                    g = gg * LANES + l
                    for p in range(HALF // LANES):
                        sl = pl.ds(p * LANES, LANES)
                        rows_v[g, sl] = rows_v[g, sl] * e
                return inner

            lax.fori_loop(0, CHUNK // LANES, srow, 0)
            pltpu.sync_copy(rows_v, acc_sh.at[dst_v], add=True)

        return carry

    lax.fori_loop(0, CHUNKS_PER_TILE, chunk_body, 0)
    plsc.subcore_barrier()
    pltpu.sync_copy(acc_sh.at[pl.ds(start, ROWS_MAIN)],
                    out_hbm.at[c, pl.ds(start, ROWS_MAIN)])

    @pl.when(s == NS - 1)
    def _():
        pltpu.sync_copy(
            acc_sh.at[pl.ds(NS * ROWS_MAIN, N_NODES - NS * ROWS_MAIN)],
            out_hbm.at[c, pl.ds(NS * ROWS_MAIN, N_NODES - NS * ROWS_MAIN)])


_sc_agg = functools.partial(
    pl.kernel,
    out_type=jax.ShapeDtypeStruct((NC, N_NODES, HALF), jnp.float32),
    mesh=plsc.VectorSubcoreMesh(core_axis_name="c", subcore_axis_name="s"),
    scratch_types=[
        pltpu.VMEM((CHUNK,), jnp.int32),
        pltpu.VMEM((CHUNK,), jnp.int32),
        pltpu.VMEM((CHUNK,), jnp.float32),
        pltpu.VMEM((CHUNK, HALF), jnp.float32),
        pltpu.VMEM_SHARED((N_NODES, HALF), jnp.float32),
        pltpu.SemaphoreType.DMA,
    ],
)(_sc_agg_body)


def _mm_body(a0_ref, a1_ref, w0_ref, w1_ref, o_ref):
    acc = jnp.dot(a0_ref[...], w0_ref[...],
                  preferred_element_type=jnp.float32,
                  precision=lax.Precision.HIGHEST)
    acc = acc + jnp.dot(a1_ref[...], w1_ref[...],
                        preferred_element_type=jnp.float32,
                        precision=lax.Precision.HIGHEST)
    o_ref[...] = jnp.maximum(acc, 0.0)


M_BLK = 1000


def _mm_relu(agg2, w):
    return pl.pallas_call(
        _mm_body,
        grid=(N_NODES // M_BLK,),
        in_specs=[
            pl.BlockSpec((M_BLK, HALF), lambda i: (i, 0)),
            pl.BlockSpec((M_BLK, HALF), lambda i: (i, 0)),
            pl.BlockSpec((HALF, D_OUT), lambda i: (0, 0)),
            pl.BlockSpec((HALF, D_OUT), lambda i: (0, 0)),
        ],
        out_specs=pl.BlockSpec((M_BLK, D_OUT), lambda i: (i, 0)),
        out_shape=jax.ShapeDtypeStruct((N_NODES, D_OUT), jnp.float32),
    )(agg2[0], agg2[1], w[:HALF], w[HALF:])


def kernel(x, edge_index, edge_values, W):
    x2 = x.reshape(2 * N_NODES, HALF)
    agg2 = _sc_agg(x2, edge_index, edge_values)
    return _mm_relu(agg2, W)
